# Initial kernel scaffold; baseline (speedup 1.0000x reference)
import jax
import jax.numpy as jnp
from jax.experimental import pallas as pl


def kernel(x, router_logits, w13, w2):
    raise NotImplementedError("write your pallas kernel here")


# speedup vs baseline: 1.0595x; 1.0595x over previous
import functools

import jax
import jax.numpy as jnp
from jax.experimental import pallas as pl
from jax.experimental.pallas import tpu as pltpu

_NUM_EXPERTS = 64
_TOP_K = 2
_HIDDEN = 1024
_INTER = 512
_TOKENS = 32
_P = _TOKENS * _TOP_K


def _routing_kernel(logits_ref, eid_ref, tok_ref, w_ref):
    l = logits_ref[...]
    T, E = l.shape
    col = jax.lax.broadcasted_iota(jnp.int32, (T, E), 1).astype(jnp.float32)

    m1 = jnp.max(l, axis=1, keepdims=True)
    a1 = jnp.min(jnp.where(l >= m1, col, jnp.float32(E)), axis=1, keepdims=True)
    lm = jnp.where(col == a1, -jnp.inf, l)
    m2 = jnp.max(lm, axis=1, keepdims=True)
    a2 = jnp.min(jnp.where(lm >= m2, col, jnp.float32(E)), axis=1, keepdims=True)

    w1 = 1.0 / (1.0 + jnp.exp(m2 - m1))
    w2 = 1.0 - w1

    tok_iota = jax.lax.broadcasted_iota(jnp.int32, (T, 1), 0).astype(jnp.float32)
    eid_col = jnp.concatenate([a1, a2], axis=0)
    tok_col = jnp.concatenate([tok_iota, tok_iota], axis=0)
    w_col = jnp.concatenate([w1, w2], axis=0)

    P = 2 * T
    pair_iota = jax.lax.broadcasted_iota(jnp.int32, (P, 1), 0).astype(jnp.float32)
    c_col = eid_col * P + pair_iota

    A = jnp.broadcast_to(c_col, (P, P))
    B = jnp.transpose(A)
    rank_col = jnp.sum((B < A).astype(jnp.float32), axis=1, keepdims=True)

    R = jnp.transpose(jnp.broadcast_to(rank_col, (P, P)))
    p_iota = jax.lax.broadcasted_iota(jnp.int32, (P, P), 0).astype(jnp.float32)
    S = (R == p_iota).astype(jnp.float32)

    dot = functools.partial(
        jax.lax.dot, precision=jax.lax.Precision.HIGHEST,
        preferred_element_type=jnp.float32)
    eid_ref[...] = dot(S, eid_col).astype(jnp.int32)
    tok_ref[...] = dot(S, tok_col).astype(jnp.int32)
    w_ref[...] = dot(S, w_col)


def _moe_kernel(eid_s, tok_s, w_s, x_ref, w13_ref, w2_ref, out_ref):
    p = pl.program_id(0)

    @pl.when(p == 0)
    def _():
        out_ref[...] = jnp.zeros_like(out_ref)

    tok = tok_s[p]
    xrow = x_ref[pl.ds(tok, 1), :]
    w13e = w13_ref[0]
    gu = jax.lax.dot_general(
        xrow, w13e, (((1,), (1,)), ((), ())),
        preferred_element_type=jnp.float32,
        precision=jax.lax.Precision.HIGHEST)
    gate = gu[:, :_INTER]
    up = gu[:, _INTER:]
    inter = gate * jax.lax.logistic(gate) * up
    w2e = w2_ref[0]
    down = jax.lax.dot_general(
        inter, w2e, (((1,), (1,)), ((), ())),
        preferred_element_type=jnp.float32,
        precision=jax.lax.Precision.HIGHEST)
    wgt = w_s[p]
    out_ref[pl.ds(tok, 1), :] = out_ref[pl.ds(tok, 1), :] + wgt * down


def kernel(x, router_logits, w13, w2):
    eid_c, tok_c, w_c = pl.pallas_call(
        _routing_kernel,
        out_shape=[
            jax.ShapeDtypeStruct((_P, 1), jnp.int32),
            jax.ShapeDtypeStruct((_P, 1), jnp.int32),
            jax.ShapeDtypeStruct((_P, 1), jnp.float32),
        ],
    )(router_logits.astype(jnp.float32))
    eid = eid_c.reshape(-1)
    tok = tok_c.reshape(-1)
    wgt = w_c.reshape(-1)

    grid_spec = pltpu.PrefetchScalarGridSpec(
        num_scalar_prefetch=3,
        grid=(_P,),
        in_specs=[
            pl.BlockSpec((_TOKENS, _HIDDEN), lambda p, e, t, w: (0, 0)),
            pl.BlockSpec((1, 2 * _INTER, _HIDDEN), lambda p, e, t, w: (e[p], 0, 0)),
            pl.BlockSpec((1, _HIDDEN, _INTER), lambda p, e, t, w: (e[p], 0, 0)),
        ],
        out_specs=pl.BlockSpec((_TOKENS, _HIDDEN), lambda p, e, t, w: (0, 0)),
    )
    out = pl.pallas_call(
        _moe_kernel,
        grid_spec=grid_spec,
        out_shape=jax.ShapeDtypeStruct((_TOKENS, _HIDDEN), jnp.float32),
        compiler_params=pltpu.CompilerParams(
            dimension_semantics=("arbitrary",)),
    )(eid, tok, wgt, x, w13, w2)
    return out.astype(x.dtype)


# speedup vs baseline: 2.7196x; 2.5668x over previous
import functools

import jax
import jax.numpy as jnp
from jax.experimental import pallas as pl
from jax.experimental.pallas import tpu as pltpu

_NUM_EXPERTS = 64
_TOP_K = 2
_HIDDEN = 1024
_INTER = 512
_TOKENS = 32
_P = _TOKENS * _TOP_K


def _routing_kernel(logits_ref, eid_ref, tok_ref, w_ref):
    l = logits_ref[...]
    T, E = l.shape
    col = jax.lax.broadcasted_iota(jnp.int32, (T, E), 1).astype(jnp.float32)

    m1 = jnp.max(l, axis=1, keepdims=True)
    a1 = jnp.min(jnp.where(l >= m1, col, jnp.float32(E)), axis=1, keepdims=True)
    lm = jnp.where(col == a1, -jnp.inf, l)
    m2 = jnp.max(lm, axis=1, keepdims=True)
    a2 = jnp.min(jnp.where(lm >= m2, col, jnp.float32(E)), axis=1, keepdims=True)

    w1 = 1.0 / (1.0 + jnp.exp(m2 - m1))
    w2 = 1.0 - w1

    tok_iota = jax.lax.broadcasted_iota(jnp.int32, (T, 1), 0).astype(jnp.float32)
    eid_col = jnp.concatenate([a1, a2], axis=0)
    tok_col = jnp.concatenate([tok_iota, tok_iota], axis=0)
    w_col = jnp.concatenate([w1, w2], axis=0)

    P = 2 * T
    pair_iota = jax.lax.broadcasted_iota(jnp.int32, (P, 1), 0).astype(jnp.float32)
    c_col = eid_col * P + pair_iota

    A = jnp.broadcast_to(c_col, (P, P))
    B = jnp.transpose(A)
    rank_col = jnp.sum((B < A).astype(jnp.float32), axis=1, keepdims=True)

    R = jnp.transpose(jnp.broadcast_to(rank_col, (P, P)))
    p_iota = jax.lax.broadcasted_iota(jnp.int32, (P, P), 0).astype(jnp.float32)
    S = (R == p_iota).astype(jnp.float32)

    dot = functools.partial(
        jax.lax.dot, precision=jax.lax.Precision.HIGHEST,
        preferred_element_type=jnp.float32)
    eid_ref[...] = dot(S, eid_col).astype(jnp.int32)
    tok_ref[...] = dot(S, tok_col).astype(jnp.int32)
    w_ref[...] = dot(S, w_col)


def _moe_kernel(eid_s, tok_s, w_s, x_ref, w13_ref, w2_ref, out_ref):
    p = pl.program_id(0)

    @pl.when(p == 0)
    def _():
        out_ref[...] = jnp.zeros_like(out_ref)

    tok = tok_s[p]
    xrow = x_ref[pl.ds(tok, 1), :]
    w13e = w13_ref[0]
    gu = jax.lax.dot_general(
        xrow, w13e, (((1,), (1,)), ((), ())),
        preferred_element_type=jnp.float32)
    gate = gu[:, :_INTER]
    up = gu[:, _INTER:]
    inter = gate * jax.lax.logistic(gate) * up
    w2e = w2_ref[0]
    down = jax.lax.dot_general(
        inter, w2e, (((1,), (1,)), ((), ())),
        preferred_element_type=jnp.float32)
    wgt = w_s[p]
    out_ref[pl.ds(tok, 1), :] = out_ref[pl.ds(tok, 1), :] + wgt * down


def kernel(x, router_logits, w13, w2):
    eid_c, tok_c, w_c = pl.pallas_call(
        _routing_kernel,
        out_shape=[
            jax.ShapeDtypeStruct((_P, 1), jnp.int32),
            jax.ShapeDtypeStruct((_P, 1), jnp.int32),
            jax.ShapeDtypeStruct((_P, 1), jnp.float32),
        ],
    )(router_logits.astype(jnp.float32))
    eid = eid_c.reshape(-1)
    tok = tok_c.reshape(-1)
    wgt = w_c.reshape(-1)

    grid_spec = pltpu.PrefetchScalarGridSpec(
        num_scalar_prefetch=3,
        grid=(_P,),
        in_specs=[
            pl.BlockSpec((_TOKENS, _HIDDEN), lambda p, e, t, w: (0, 0)),
            pl.BlockSpec((1, 2 * _INTER, _HIDDEN), lambda p, e, t, w: (e[p], 0, 0)),
            pl.BlockSpec((1, _HIDDEN, _INTER), lambda p, e, t, w: (e[p], 0, 0)),
        ],
        out_specs=pl.BlockSpec((_TOKENS, _HIDDEN), lambda p, e, t, w: (0, 0)),
    )
    out = pl.pallas_call(
        _moe_kernel,
        grid_spec=grid_spec,
        out_shape=jax.ShapeDtypeStruct((_TOKENS, _HIDDEN), jnp.float32),
        compiler_params=pltpu.CompilerParams(
            dimension_semantics=("arbitrary",)),
    )(eid, tok, wgt, x, w13, w2)
    return out.astype(x.dtype)
